# speedup vs baseline: 18.4575x; 18.4575x over previous
import functools

import jax
import jax.numpy as jnp
from jax.experimental import pallas as pl

_ITERS = 26


def _sparsemax_block(x_ref, o_ref):
    x = x_ref[...]
    m = jnp.max(x, axis=-1, keepdims=True)
    lo = m - 1.0
    hi = m
    mask = x > lo
    k = jnp.sum(mask.astype(jnp.float32), axis=-1, keepdims=True)
    s = jnp.sum(jnp.where(mask, x, 0.0), axis=-1, keepdims=True)

    def body(_, carry):
        lo, hi, k, s = carry
        newton = (s - 1.0) / k
        mid = 0.5 * (lo + hi)
        t = jnp.maximum(newton, mid)
        maskt = x > t
        kt = jnp.sum(maskt.astype(jnp.float32), axis=-1, keepdims=True)
        st = jnp.sum(jnp.where(maskt, x, 0.0), axis=-1, keepdims=True)
        ft = st - kt * t - 1.0
        good = ft >= 0.0
        lo = jnp.where(good, t, lo)
        hi = jnp.where(good, hi, t)
        k = jnp.where(good, kt, k)
        s = jnp.where(good, st, s)
        return lo, hi, k, s

    lo, hi, k, s = jax.lax.fori_loop(0, _ITERS, body, (lo, hi, k, s))
    tau = (s - 1.0) / k
    o_ref[...] = jnp.maximum(x - tau, 0.0)


@functools.partial(jax.jit, static_argnames=("block_rows",))
def _sparsemax(x, block_rows=8):
    rows, n = x.shape
    grid = (rows // block_rows,)
    return pl.pallas_call(
        _sparsemax_block,
        grid=grid,
        in_specs=[pl.BlockSpec((block_rows, n), lambda i: (i, 0))],
        out_specs=pl.BlockSpec((block_rows, n), lambda i: (i, 0)),
        out_shape=jax.ShapeDtypeStruct((rows, n), x.dtype),
    )(x)


def kernel(x):
    return _sparsemax(x)


# speedup vs baseline: 23.9673x; 1.2985x over previous
import functools

import jax
import jax.numpy as jnp
from jax.experimental import pallas as pl

_ITERS = 26


def _sparsemax_block(x_ref, o_ref):
    x = x_ref[...]
    m = jnp.max(x, axis=-1, keepdims=True)
    lo = m - 1.0
    hi = m
    mask = x > lo
    k = jnp.sum(mask.astype(jnp.float32), axis=-1, keepdims=True)
    s = jnp.sum(jnp.where(mask, x, 0.0), axis=-1, keepdims=True)

    def cond(carry):
        i, lo, hi, k, s = carry
        return jnp.logical_and(i < _ITERS, jnp.any((s - 1.0) / k > lo))

    def body(carry):
        i, lo, hi, k, s = carry
        newton = (s - 1.0) / k
        mid = 0.5 * (lo + hi)
        t = jnp.maximum(newton, mid)
        maskt = x > t
        kt = jnp.sum(maskt.astype(jnp.float32), axis=-1, keepdims=True)
        st = jnp.sum(jnp.where(maskt, x, 0.0), axis=-1, keepdims=True)
        ft = st - kt * t - 1.0
        good = ft >= 0.0
        lo = jnp.where(good, t, lo)
        hi = jnp.where(good, hi, t)
        k = jnp.where(good, kt, k)
        s = jnp.where(good, st, s)
        return i + 1, lo, hi, k, s

    _, lo, hi, k, s = jax.lax.while_loop(cond, body, (0, lo, hi, k, s))
    tau = (s - 1.0) / k
    o_ref[...] = jnp.maximum(x - tau, 0.0)


@functools.partial(jax.jit, static_argnames=("block_rows",))
def _sparsemax(x, block_rows=8):
    rows, n = x.shape
    grid = (rows // block_rows,)
    return pl.pallas_call(
        _sparsemax_block,
        grid=grid,
        in_specs=[pl.BlockSpec((block_rows, n), lambda i: (i, 0))],
        out_specs=pl.BlockSpec((block_rows, n), lambda i: (i, 0)),
        out_shape=jax.ShapeDtypeStruct((rows, n), x.dtype),
    )(x)


def kernel(x):
    return _sparsemax(x)
